# baseline (speedup 1.0000x reference)
import functools

import jax
import jax.numpy as jnp
from jax import lax
from jax.experimental import pallas as pl
from jax.experimental.pallas import tpu as pltpu
from jax.experimental.pallas import tpu_sc as plsc

_NC = 2
_NS = 16
_L = 16
_NW = _NC * _NS
_RPW = 4


def _sc_hist_body(cp, num_classes,
                  yp_hbm, yt_hbm, counts_out, soft_out, o_out,
                  yp_v, yt_v, f_v, cl_v, pm_v, pp_v,
                  ones_v, zero_v, o_red, sh_cnt, sh_sf, sh_scl, sem):
    cid = lax.axis_index("c")
    sid = lax.axis_index("s")
    wid = sid * _NC + cid
    row0 = wid * _RPW

    zeros16 = jnp.zeros((_L,), jnp.float32)
    ones16 = jnp.ones((_L,), jnp.float32)

    @pl.when(sid == 0)
    def _zero_spmem():
        def zb(j, c):
            zero_v[pl.ds(j * _L, _L)] = zeros16
            return c
        lax.fori_loop(0, cp // _L, zb, 0)
        pltpu.sync_copy(zero_v, sh_cnt)
        pltpu.sync_copy(zero_v, sh_sf)
        pltpu.sync_copy(zero_v, sh_scl)

    pltpu.sync_copy(yp_hbm.at[pl.ds(row0, _RPW)], yp_v)
    pltpu.sync_copy(yt_hbm.at[pl.ds(row0, _RPW)], yt_v)
    for j in range(128 // _L):
        ones_v[pl.ds(j * _L, _L)] = ones16

    o_acc = zeros16
    for r in range(_RPW):
        for k in range(128 // _L):
            sl = pl.ds(k * _L, _L)
            yp = yp_v[r, sl]
            yt = yt_v[r, sl]
            d = yp - yt.astype(jnp.float32)
            o_acc = o_acc + d * d
            ypc = jnp.clip(yp, 0.0, float(num_classes - 1))
            f = ypc.astype(jnp.int32)
            p = ypc - f.astype(jnp.float32)
            f_v[r, sl] = f
            cl_v[r, sl] = f + (p > 0.0).astype(jnp.int32)
            pm_v[r, sl] = ones16 - p
            pp_v[r, sl] = p
    o_red[...] = o_acc
    pltpu.sync_copy(o_red, o_out.at[pl.ds(wid * _L, _L)])

    plsc.subcore_barrier()

    for j in range(_RPW):
        d1 = pltpu.async_copy(ones_v, sh_cnt.at[yt_v.at[j]], sem, add=True)
        d2 = pltpu.async_copy(pm_v.at[j], sh_sf.at[f_v.at[j]], sem, add=True)
        d3 = pltpu.async_copy(pp_v.at[j], sh_scl.at[cl_v.at[j]], sem, add=True)
        d1.wait()
        d2.wait()
        d3.wait()

    plsc.subcore_barrier()

    @pl.when(sid == 0)
    def _write_out():
        pltpu.sync_copy(sh_cnt, counts_out.at[cid])
        pltpu.sync_copy(sh_sf, soft_out.at[cid])
        pltpu.sync_copy(sh_scl, soft_out.at[cid + _NC])


def _make_sc_hist(batch, cp, num_classes):
    assert batch == _NW * _RPW * 128
    mesh = plsc.VectorSubcoreMesh(core_axis_name="c", subcore_axis_name="s")
    return functools.partial(
        pl.kernel,
        mesh=mesh,
        compiler_params=pltpu.CompilerParams(use_tc_tiling_on_sc=False,
                                             needs_layout_passes=False),
        out_type=(
            jax.ShapeDtypeStruct((_NC, cp), jnp.float32),
            jax.ShapeDtypeStruct((2 * _NC, cp), jnp.float32),
            jax.ShapeDtypeStruct((_NW * _L,), jnp.float32),
        ),
        scratch_types=[
            pltpu.VMEM((_RPW, 128), jnp.float32),
            pltpu.VMEM((_RPW, 128), jnp.int32),
            pltpu.VMEM((_RPW, 128), jnp.int32),
            pltpu.VMEM((_RPW, 128), jnp.int32),
            pltpu.VMEM((_RPW, 128), jnp.float32),
            pltpu.VMEM((_RPW, 128), jnp.float32),
            pltpu.VMEM((128,), jnp.float32),
            pltpu.VMEM((cp,), jnp.float32),
            pltpu.VMEM((_L,), jnp.float32),
            pltpu.VMEM_SHARED((cp,), jnp.float32),
            pltpu.VMEM_SHARED((cp,), jnp.float32),
            pltpu.VMEM_SHARED((cp,), jnp.float32),
            pltpu.SemaphoreType.DMA,
        ],
    )(functools.partial(_sc_hist_body, cp, num_classes))


def _combine_body(batch, num_classes, eps,
                  counts_ref, soft_ref, o_ref, hb_ref, w_ref, out_ref):
    counts = jnp.sum(counts_ref[...], axis=0, keepdims=True)
    soft = jnp.sum(soft_ref[...], axis=0, keepdims=True)
    o_total = jnp.sum(o_ref[...])
    hi = jax.lax.Precision.HIGHEST
    ht = jnp.dot(counts[:, :num_classes], hb_ref[...], precision=hi,
                 preferred_element_type=jnp.float32)
    hp = jnp.dot(soft[:, :num_classes], hb_ref[...], precision=hi,
                 preferred_element_type=jnp.float32)
    t = jnp.dot(ht, w_ref[...], precision=hi,
                preferred_element_type=jnp.float32)
    e = jnp.sum(t * hp) / float(batch)
    out_ref[...] = jnp.log(o_total / (e + eps)).reshape(1, 1)


def _combine(counts_p, soft_p, o_p, hist_bricks, weights, batch, eps):
    num_classes = hist_bricks.shape[0]
    body = functools.partial(_combine_body, batch, num_classes, eps)
    out = pl.pallas_call(
        body,
        out_shape=jax.ShapeDtypeStruct((1, 1), jnp.float32),
    )(counts_p, soft_p, o_p, hist_bricks, weights)
    return out[0, 0]


def kernel(y_pred, y_true, weights, hist_bricks):
    batch = y_pred.shape[0]
    num_classes = hist_bricks.shape[0]
    cp = ((num_classes + 127) // 128) * 128

    yp2 = y_pred.reshape(batch // 128, 128)
    yt2 = y_true.reshape(batch // 128, 128).astype(jnp.int32)

    counts_p, soft_p, o_p = _make_sc_hist(batch, cp, num_classes)(yp2, yt2)
    o_p = o_p.reshape(4, (_NW * _L) // 4)
    return _combine(counts_p, soft_p, o_p, hist_bricks, weights,
                    batch, 1e-10)


# speedup vs baseline: 1.2577x; 1.2577x over previous
import functools

import jax
import jax.numpy as jnp
from jax import lax
from jax.experimental import pallas as pl
from jax.experimental.pallas import tpu as pltpu
from jax.experimental.pallas import tpu_sc as plsc

_NC = 2
_NS = 16
_L = 16
_NW = _NC * _NS


def _sc_hist_body(cp, chunk, num_classes,
                  y_pred_hbm, y_true_hbm, counts_out, soft_out, o_out,
                  yp_v, yt_v, cnt_rows, soft_rows, cnt_red, soft_red, o_red):
    wid = lax.axis_index("s") * _NC + lax.axis_index("c")
    base = wid * chunk

    with jax.named_scope("ph_load"):
        pltpu.sync_copy(y_pred_hbm.at[pl.ds(base, chunk)], yp_v)
        pltpu.sync_copy(y_true_hbm.at[pl.ds(base, chunk)], yt_v)

    zeros16 = jnp.zeros((_L,), jnp.float32)
    ones16 = jnp.ones((_L,), jnp.float32)
    lane = lax.iota(jnp.int32, _L)

    def zero_body(j, c):
        for r in range(_L):
            cnt_rows[r, pl.ds(j * _L, _L)] = zeros16
            soft_rows[r, pl.ds(j * _L, _L)] = zeros16
        return c
    with jax.named_scope("ph_zero"):
        lax.fori_loop(0, cp // _L, zero_body, 0)

    def hist_body(g, o_acc):
        yp = yp_v[pl.ds(g * _L, _L)]
        yt = yt_v[pl.ds(g * _L, _L)]
        d = yp - yt.astype(jnp.float32)
        o_acc = o_acc + d * d
        plsc.addupdate_scatter(cnt_rows, [lane, yt], ones16)
        ypc = jnp.clip(yp, 0.0, float(num_classes - 1))
        f = ypc.astype(jnp.int32)
        p = ypc - f.astype(jnp.float32)
        cl = f + (p > 0.0).astype(jnp.int32)
        plsc.addupdate_scatter(soft_rows, [lane, f], ones16 - p)
        plsc.addupdate_scatter(soft_rows, [lane, cl], p)
        return o_acc
    with jax.named_scope("ph_hist"):
        o_acc = lax.fori_loop(0, chunk // _L, hist_body,
                              jnp.zeros((_L,), jnp.float32))
    o_red[...] = o_acc

    def red_body(j, c):
        ca = cnt_rows[0, pl.ds(j * _L, _L)]
        sa = soft_rows[0, pl.ds(j * _L, _L)]
        for r in range(1, _L):
            ca = ca + cnt_rows[r, pl.ds(j * _L, _L)]
            sa = sa + soft_rows[r, pl.ds(j * _L, _L)]
        cnt_red[pl.ds(j * _L, _L)] = ca
        soft_red[pl.ds(j * _L, _L)] = sa
        return c
    with jax.named_scope("ph_reduce"):
        lax.fori_loop(0, cp // _L, red_body, 0)

    with jax.named_scope("ph_writeout"):
        pltpu.sync_copy(cnt_red, counts_out.at[wid])
        pltpu.sync_copy(soft_red, soft_out.at[wid])
        pltpu.sync_copy(o_red, o_out.at[pl.ds(wid * _L, _L)])


def _make_sc_hist(batch, cp, num_classes):
    chunk = batch // _NW
    mesh = plsc.VectorSubcoreMesh(core_axis_name="c", subcore_axis_name="s")
    return functools.partial(
        pl.kernel,
        mesh=mesh,
        compiler_params=pltpu.CompilerParams(use_tc_tiling_on_sc=False,
                                             needs_layout_passes=False),
        out_type=(
            jax.ShapeDtypeStruct((_NW, cp), jnp.float32),
            jax.ShapeDtypeStruct((_NW, cp), jnp.float32),
            jax.ShapeDtypeStruct((_NW * _L,), jnp.float32),
        ),
        scratch_types=[
            pltpu.VMEM((chunk,), jnp.float32),
            pltpu.VMEM((chunk,), jnp.int32),
            pltpu.VMEM((_L, cp), jnp.float32),
            pltpu.VMEM((_L, cp), jnp.float32),
            pltpu.VMEM((cp,), jnp.float32),
            pltpu.VMEM((cp,), jnp.float32),
            pltpu.VMEM((_L,), jnp.float32),
        ],
    )(functools.partial(_sc_hist_body, cp, chunk, num_classes))


def _combine_body(batch, num_classes, eps,
                  counts_ref, soft_ref, o_ref, hb_ref, w_ref, out_ref):
    counts = jnp.sum(counts_ref[...], axis=0, keepdims=True)
    soft = jnp.sum(soft_ref[...], axis=0, keepdims=True)
    o_total = jnp.sum(o_ref[...])
    hi = jax.lax.Precision.HIGHEST
    ht = jnp.dot(counts[:, :num_classes], hb_ref[...], precision=hi,
                 preferred_element_type=jnp.float32)
    hp = jnp.dot(soft[:, :num_classes], hb_ref[...], precision=hi,
                 preferred_element_type=jnp.float32)
    t = jnp.dot(ht, w_ref[...], precision=hi,
                preferred_element_type=jnp.float32)
    e = jnp.sum(t * hp) / float(batch)
    out_ref[...] = jnp.log(o_total / (e + eps)).reshape(1, 1)


def _combine(counts_p, soft_p, o_p, hist_bricks, weights, batch, eps):
    num_classes = hist_bricks.shape[0]
    body = functools.partial(_combine_body, batch, num_classes, eps)
    out = pl.pallas_call(
        body,
        out_shape=jax.ShapeDtypeStruct((1, 1), jnp.float32),
    )(counts_p, soft_p, o_p, hist_bricks, weights)
    return out[0, 0]


def kernel(y_pred, y_true, weights, hist_bricks):
    batch = y_pred.shape[0]
    num_classes = hist_bricks.shape[0]
    cp = ((num_classes + 127) // 128) * 128

    ypf = y_pred.reshape(batch)
    yti = y_true.reshape(batch).astype(jnp.int32)

    counts_p, soft_p, o_p = _make_sc_hist(batch, cp, num_classes)(ypf, yti)
    o_p = o_p.reshape(4, (_NW * _L) // 4)
    return _combine(counts_p, soft_p, o_p, hist_bricks, weights,
                    batch, 1e-10)
